# baseline (speedup 1.0000x reference)
import functools

import jax
import jax.numpy as jnp
from jax import lax
from jax.experimental import pallas as pl
from jax.experimental.pallas import tpu as pltpu
from jax.experimental.pallas import tpu_sc as plsc

N = 10000
D = 128
E = 320000
NC, NS, L = 2, 16, 16
NW = NC * NS
NPAD = 10240
CHUNK = 96
BLK = 15
NBLK = 7
TCH = NBLK * BLK
TE = TCH * CHUNK
EPAD = NW * TE
RPT = NPAD // NS
NB = 1280
GRID = NPAD // NB


def _deg_body(src_hbm, dst_hbm, out_hbm, sidx_v, didx_v, hist_s, hist_d):
    c = lax.axis_index("c")
    s = lax.axis_index("s")
    w = c * NS + s
    z16 = jnp.zeros((L,), jnp.float32)

    @pl.loop(0, NPAD // L)
    def _zero(i):
        hist_s[pl.ds(i * L, L)] = z16
        hist_d[pl.ds(i * L, L)] = z16

    pltpu.sync_copy(src_hbm.at[pl.ds(w * TE, TE)], sidx_v)
    pltpu.sync_copy(dst_hbm.at[pl.ds(w * TE, TE)], didx_v)
    ones16 = jnp.ones((L,), jnp.float32)

    @pl.loop(0, TE // L)
    def _hist(i):
        plsc.addupdate_scatter(hist_s, [sidx_v[pl.ds(i * L, L)]], ones16)
        plsc.addupdate_scatter(hist_d, [didx_v[pl.ds(i * L, L)]], ones16)

    pltpu.sync_copy(hist_s, out_hbm.at[w, 0])
    pltpu.sync_copy(hist_d, out_hbm.at[w, 1])


@jax.jit
def _deg_call(src1, dst1):
    mesh = plsc.VectorSubcoreMesh(core_axis_name="c", subcore_axis_name="s",
                                  num_cores=NC, num_subcores=NS)
    return pl.kernel(
        _deg_body,
        out_type=jax.ShapeDtypeStruct((NW, 2, NPAD), jnp.float32),
        mesh=mesh,
        scratch_types=[
            pltpu.VMEM((TE,), jnp.int32),
            pltpu.VMEM((TE,), jnp.int32),
            pltpu.VMEM((NPAD,), jnp.float32),
            pltpu.VMEM((NPAD,), jnp.float32),
        ],
        compiler_params=pltpu.CompilerParams(needs_layout_passes=False),
    )(src1, dst1)


def _msg_body(g_hbm, src_hbm, dst_hbm, out_hbm, sa_v, da_v, sb_v, db_v,
              r0_v, r1_v, r2_v, acc_sh, s0, s1, s2, ia, ib):
    c = lax.axis_index("c")
    s = lax.axis_index("s")
    w = c * NS + s
    z16 = jnp.zeros((L,), jnp.float32)
    rings = (r0_v, r1_v, r2_v)
    rsems = (s0, s1, s2)

    @pl.loop(0, CHUNK)
    def _zrow(r):
        for k in range(D // L):
            r0_v[r, pl.ds(k * L, L)] = z16

    for j in range(RPT // CHUNK):
        pltpu.sync_copy(r0_v, acc_sh.at[pl.ds(s * RPT + j * CHUNK, CHUNK)])
    rem = RPT - (RPT // CHUNK) * CHUNK
    if rem:
        pltpu.sync_copy(r0_v.at[pl.ds(0, rem)],
                        acc_sh.at[pl.ds(s * RPT + (RPT // CHUNK) * CHUNK, rem)])

    plsc.subcore_barrier()

    pltpu.sync_copy(src_hbm.at[w, 0], sa_v)
    pltpu.sync_copy(dst_hbm.at[w, 0], da_v)
    pltpu.async_copy(src_hbm.at[w, 1], sb_v, ib)
    pltpu.async_copy(dst_hbm.at[w, 1], db_v, ib)
    for q in range(3):
        pltpu.async_copy(g_hbm.at[sa_v.at[q]], rings[q], rsems[q])

    def _block(s_cur, d_cur, s_nxt, d_nxt, sem_nxt, sem_pref, pref_b):
        @pl.loop(0, 4)
        def _grp(g):
            for q in range(3):
                j = 3 * g + q
                pltpu.make_async_copy(g_hbm.at[s_cur.at[j]], rings[q],
                                      rsems[q]).wait()
                pltpu.sync_copy(rings[q], acc_sh.at[d_cur.at[j]], add=True)
                pltpu.async_copy(g_hbm.at[s_cur.at[j + 3]], rings[q], rsems[q])

        pltpu.make_async_copy(src_hbm.at[w, 0], s_nxt, sem_nxt).wait()
        pltpu.make_async_copy(dst_hbm.at[w, 0], d_nxt, sem_nxt).wait()
        for q in range(3):
            j = BLK - 3 + q
            pltpu.make_async_copy(g_hbm.at[s_cur.at[j]], rings[q],
                                  rsems[q]).wait()
            pltpu.sync_copy(rings[q], acc_sh.at[d_cur.at[j]], add=True)
            pltpu.async_copy(g_hbm.at[s_nxt.at[q]], rings[q], rsems[q])
        pltpu.async_copy(src_hbm.at[w, pref_b], s_cur, sem_pref)
        pltpu.async_copy(dst_hbm.at[w, pref_b], d_cur, sem_pref)

    @pl.loop(0, (NBLK - 1) // 2)
    def _pair(p):
        _block(sa_v, da_v, sb_v, db_v, ib, ia, 2 * p + 2)
        _block(sb_v, db_v, sa_v, da_v, ia, ib, 2 * p + 3)

    @pl.loop(0, 4)
    def _grp_f(g):
        for q in range(3):
            j = 3 * g + q
            pltpu.make_async_copy(g_hbm.at[sa_v.at[j]], rings[q],
                                  rsems[q]).wait()
            pltpu.sync_copy(rings[q], acc_sh.at[da_v.at[j]], add=True)
            pltpu.async_copy(g_hbm.at[sa_v.at[j + 3]], rings[q], rsems[q])

    for q in range(3):
        j = BLK - 3 + q
        pltpu.make_async_copy(g_hbm.at[sa_v.at[j]], rings[q], rsems[q]).wait()
        pltpu.sync_copy(rings[q], acc_sh.at[da_v.at[j]], add=True)
    pltpu.make_async_copy(src_hbm.at[w, 0], sb_v, ib).wait()
    pltpu.make_async_copy(dst_hbm.at[w, 0], db_v, ib).wait()

    plsc.subcore_barrier()
    pltpu.sync_copy(acc_sh.at[pl.ds(s * RPT, RPT)],
                    out_hbm.at[c, pl.ds(s * RPT, RPT)])


@jax.jit
def _msg_call(g, srcR, dstR):
    mesh = plsc.VectorSubcoreMesh(core_axis_name="c", subcore_axis_name="s",
                                  num_cores=NC, num_subcores=NS)
    return pl.kernel(
        _msg_body,
        out_type=jax.ShapeDtypeStruct((NC, NPAD, D), jnp.float32),
        mesh=mesh,
        scratch_types=[
            pltpu.VMEM((BLK, CHUNK), jnp.int32),
            pltpu.VMEM((BLK, CHUNK), jnp.int32),
            pltpu.VMEM((BLK, CHUNK), jnp.int32),
            pltpu.VMEM((BLK, CHUNK), jnp.int32),
            pltpu.VMEM((CHUNK, D), jnp.float32),
            pltpu.VMEM((CHUNK, D), jnp.float32),
            pltpu.VMEM((CHUNK, D), jnp.float32),
            pltpu.VMEM_SHARED((NPAD, D), jnp.float32),
            pltpu.SemaphoreType.DMA,
            pltpu.SemaphoreType.DMA,
            pltpu.SemaphoreType.DMA,
            pltpu.SemaphoreType.DMA,
            pltpu.SemaphoreType.DMA,
        ],
    )(g, srcR, dstR)


def _tc_b_body(degp_ref, x_ref, w1_ref, g1_ref, norms_ref):
    deg = jnp.sum(degp_ref[...], axis=0)
    nrm = lax.rsqrt(jnp.where(deg > 0, deg, 1.0))
    norms_ref[...] = nrm
    ns = nrm[0][:, None]
    g1_ref[...] = jnp.dot(x_ref[...] * ns, w1_ref[...],
                          preferred_element_type=jnp.float32)


@jax.jit
def _tc_b_call(degp, x_pad, W1):
    return pl.pallas_call(
        _tc_b_body,
        grid=(GRID,),
        in_specs=[
            pl.BlockSpec((NW, 2, NB), lambda i: (0, 0, i)),
            pl.BlockSpec((NB, D), lambda i: (i, 0)),
            pl.BlockSpec((D, D), lambda i: (0, 0)),
        ],
        out_specs=[
            pl.BlockSpec((NB, D), lambda i: (i, 0)),
            pl.BlockSpec((2, NB), lambda i: (0, i)),
        ],
        out_shape=[
            jax.ShapeDtypeStruct((NPAD, D), jnp.float32),
            jax.ShapeDtypeStruct((2, NPAD), jnp.float32),
        ],
    )(degp, x_pad, W1)


def _tc_d_body(p_ref, norms_ref, b1_ref, w2_ref, g2_ref):
    i = pl.program_id(0)
    m = p_ref[0] + p_ref[1]
    nd = norms_ref[1][:, None]
    h = jnp.maximum(m * nd + b1_ref[...], 0.0)
    rows = i * NB + lax.broadcasted_iota(jnp.int32, (NB, 1), 0)
    h = jnp.where(rows < N, h, 0.0)
    ns = norms_ref[0][:, None]
    g2_ref[...] = jnp.dot(h * ns, w2_ref[...],
                          preferred_element_type=jnp.float32)


@jax.jit
def _tc_d_call(p, norms, b1r, W2):
    return pl.pallas_call(
        _tc_d_body,
        grid=(GRID,),
        in_specs=[
            pl.BlockSpec((NC, NB, D), lambda i: (0, i, 0)),
            pl.BlockSpec((2, NB), lambda i: (0, i)),
            pl.BlockSpec((1, D), lambda i: (0, 0)),
            pl.BlockSpec((D, D), lambda i: (0, 0)),
        ],
        out_specs=pl.BlockSpec((NB, D), lambda i: (i, 0)),
        out_shape=jax.ShapeDtypeStruct((NPAD, D), jnp.float32),
    )(p, norms, b1r, W2)


def _tc_f_body(p_ref, norms_ref, b2_ref, wg_ref, bg_ref, wo_ref, bo_ref,
               out_ref, m_s, s_s, r_v):
    i = pl.program_id(0)

    @pl.when(i == 0)
    def _init():
        m_s[0] = -jnp.inf
        s_s[0] = 0.0
        r_v[...] = jnp.zeros_like(r_v)

    m = p_ref[0] + p_ref[1]
    nd = norms_ref[1][:, None]
    h = jnp.maximum(m * nd + b2_ref[...], 0.0)
    rows = i * NB + lax.broadcasted_iota(jnp.int32, (NB, 1), 0)
    h = jnp.where(rows < N, h, 0.0)
    z = jnp.sum(h * wg_ref[...], axis=1, keepdims=True) + bg_ref[0, 0]
    z = jnp.where(rows < N, z, -jnp.inf)

    m_old = m_s[0]
    m_new = jnp.maximum(m_old, jnp.max(z))
    scale = jnp.exp(m_old - m_new)
    e = jnp.exp(z - m_new)
    s_s[0] = s_s[0] * scale + jnp.sum(e)
    r_v[...] = r_v[...] * scale + jnp.sum(e * h, axis=0, keepdims=True)
    m_s[0] = m_new

    @pl.when(i == pl.num_programs(0) - 1)
    def _fin():
        r = r_v[...] / s_s[0]
        val = jnp.sum(r * wo_ref[...]) + bo_ref[0, 0]
        out_ref[...] = jnp.full((8, 128), val, jnp.float32)


@jax.jit
def _tc_f_call(p, norms, b2r, wgr, bgr, wor, bor):
    return pl.pallas_call(
        _tc_f_body,
        grid=(GRID,),
        in_specs=[
            pl.BlockSpec((NC, NB, D), lambda i: (0, i, 0)),
            pl.BlockSpec((2, NB), lambda i: (0, i)),
            pl.BlockSpec((1, D), lambda i: (0, 0)),
            pl.BlockSpec((1, D), lambda i: (0, 0)),
            pl.BlockSpec((1, 1), lambda i: (0, 0)),
            pl.BlockSpec((1, D), lambda i: (0, 0)),
            pl.BlockSpec((1, 1), lambda i: (0, 0)),
        ],
        out_specs=pl.BlockSpec((8, 128), lambda i: (0, 0)),
        out_shape=jax.ShapeDtypeStruct((8, 128), jnp.float32),
        scratch_shapes=[
            pltpu.SMEM((1,), jnp.float32),
            pltpu.SMEM((1,), jnp.float32),
            pltpu.VMEM((1, D), jnp.float32),
        ],
    )(p, norms, b2r, wgr, bgr, wor, bor)


def kernel(x, edge_index, W1, b1, W2, b2, Wg, bg, Wo, bo):
    src = edge_index[0]
    dst = edge_index[1]
    pad = N + (jnp.arange(EPAD - E, dtype=jnp.int32) % (NPAD - N))
    src1 = jnp.concatenate([src, pad])
    dst1 = jnp.concatenate([dst, pad])
    dummy = jnp.full((NW, 1, BLK, CHUNK), N, jnp.int32)
    srcR = jnp.concatenate([src1.reshape(NW, NBLK, BLK, CHUNK), dummy], axis=1)
    dstR = jnp.concatenate([dst1.reshape(NW, NBLK, BLK, CHUNK), dummy], axis=1)
    x_pad = jnp.zeros((NPAD, D), jnp.float32).at[:N].set(x)

    degp = _deg_call(src1, dst1)
    g1, norms = _tc_b_call(degp, x_pad, W1)
    p1 = _msg_call(g1, srcR, dstR)
    g2 = _tc_d_call(p1, norms, b1.reshape(1, D), W2)
    p2 = _msg_call(g2, srcR, dstR)
    buf = _tc_f_call(p2, norms, b2.reshape(1, D),
                     Wg[:, 0].reshape(1, D), bg.reshape(1, 1),
                     Wo[:, 0].reshape(1, D), bo.reshape(1, 1))
    return buf[:1, :1]


# speedup vs baseline: 1.0136x; 1.0136x over previous
import functools

import jax
import jax.numpy as jnp
from jax import lax
from jax.experimental import pallas as pl
from jax.experimental.pallas import tpu as pltpu
from jax.experimental.pallas import tpu_sc as plsc

N = 10000
D = 128
E = 320000
NC, NS, L = 2, 16, 16
NW = NC * NS
NPAD = 10240
CHUNK = 80
RING = 4
BLK = 16
NBLK = 8
TCH = NBLK * BLK
TE = TCH * CHUNK
EPAD = NW * TE
IB = BLK * CHUNK
RPT = NPAD // NS
NB = 1280
GRID = NPAD // NB


def _deg_body(src_hbm, dst_hbm, out_hbm, sidx_v, didx_v, hist_s, hist_d):
    c = lax.axis_index("c")
    s = lax.axis_index("s")
    w = c * NS + s
    z16 = jnp.zeros((L,), jnp.float32)

    @pl.loop(0, NPAD // L)
    def _zero(i):
        hist_s[pl.ds(i * L, L)] = z16
        hist_d[pl.ds(i * L, L)] = z16

    pltpu.sync_copy(src_hbm.at[pl.ds(w * TE, TE)], sidx_v)
    pltpu.sync_copy(dst_hbm.at[pl.ds(w * TE, TE)], didx_v)
    ones16 = jnp.ones((L,), jnp.float32)

    @pl.loop(0, TE // L)
    def _hist(i):
        plsc.addupdate_scatter(hist_s, [sidx_v[pl.ds(i * L, L)]], ones16)
        plsc.addupdate_scatter(hist_d, [didx_v[pl.ds(i * L, L)]], ones16)

    pltpu.sync_copy(hist_s, out_hbm.at[w, 0])
    pltpu.sync_copy(hist_d, out_hbm.at[w, 1])


@jax.jit
def _deg_call(src1, dst1):
    mesh = plsc.VectorSubcoreMesh(core_axis_name="c", subcore_axis_name="s",
                                  num_cores=NC, num_subcores=NS)
    return pl.kernel(
        _deg_body,
        out_type=jax.ShapeDtypeStruct((NW, 2, NPAD), jnp.float32),
        mesh=mesh,
        scratch_types=[
            pltpu.VMEM((TE,), jnp.int32),
            pltpu.VMEM((TE,), jnp.int32),
            pltpu.VMEM((NPAD,), jnp.float32),
            pltpu.VMEM((NPAD,), jnp.float32),
        ],
        compiler_params=pltpu.CompilerParams(needs_layout_passes=False),
    )(src1, dst1)


def _msg_body(g_hbm, src_hbm, dst_hbm, out_hbm, sa_v, da_v, sb_v, db_v,
              r0_v, r1_v, r2_v, r3_v, acc_sh, s0, s1, s2, s3, ia, ib):
    c = lax.axis_index("c")
    s = lax.axis_index("s")
    w = c * NS + s
    z16 = jnp.zeros((L,), jnp.float32)
    rings = (r0_v, r1_v, r2_v, r3_v)
    rsems = (s0, s1, s2, s3)

    def _iblk(b):
        return pl.ds((w * (NBLK + 1) + b) * IB, IB)

    @pl.loop(0, CHUNK)
    def _zrow(r):
        for k in range(D // L):
            r0_v[r, pl.ds(k * L, L)] = z16

    for j in range(RPT // CHUNK):
        pltpu.sync_copy(r0_v, acc_sh.at[pl.ds(s * RPT + j * CHUNK, CHUNK)])

    plsc.subcore_barrier()

    pltpu.sync_copy(src_hbm.at[_iblk(0)], sa_v)
    pltpu.sync_copy(dst_hbm.at[_iblk(0)], da_v)
    pltpu.async_copy(src_hbm.at[_iblk(1)], sb_v, ib)
    pltpu.async_copy(dst_hbm.at[_iblk(1)], db_v, ib)
    for q in range(RING):
        pltpu.async_copy(g_hbm.at[sa_v.at[pl.ds(q * CHUNK, CHUNK)]],
                         rings[q], rsems[q])

    def _block(s_cur, d_cur, s_nxt, d_nxt, sem_nxt, sem_pref, pref_b):
        @pl.loop(0, BLK // RING - 1)
        def _grp(g):
            for q in range(RING):
                j = RING * g + q
                pltpu.make_async_copy(
                    g_hbm.at[s_cur.at[pl.ds(j * CHUNK, CHUNK)]],
                    rings[q], rsems[q]).wait()
                pltpu.sync_copy(
                    rings[q], acc_sh.at[d_cur.at[pl.ds(j * CHUNK, CHUNK)]],
                    add=True)
                pltpu.async_copy(
                    g_hbm.at[s_cur.at[pl.ds((j + RING) * CHUNK, CHUNK)]],
                    rings[q], rsems[q])

        pltpu.make_async_copy(src_hbm.at[_iblk(0)], s_nxt, sem_nxt).wait()
        pltpu.make_async_copy(dst_hbm.at[_iblk(0)], d_nxt, sem_nxt).wait()
        for q in range(RING):
            j = BLK - RING + q
            pltpu.make_async_copy(
                g_hbm.at[s_cur.at[pl.ds(j * CHUNK, CHUNK)]],
                rings[q], rsems[q]).wait()
            pltpu.sync_copy(
                rings[q], acc_sh.at[d_cur.at[pl.ds(j * CHUNK, CHUNK)]],
                add=True)
            pltpu.async_copy(g_hbm.at[s_nxt.at[pl.ds(q * CHUNK, CHUNK)]],
                             rings[q], rsems[q])
        pltpu.async_copy(src_hbm.at[_iblk(pref_b)], s_cur, sem_pref)
        pltpu.async_copy(dst_hbm.at[_iblk(pref_b)], d_cur, sem_pref)

    @pl.loop(0, NBLK // 2 - 1)
    def _pair(p):
        _block(sa_v, da_v, sb_v, db_v, ib, ia, 2 * p + 2)
        _block(sb_v, db_v, sa_v, da_v, ia, ib, 2 * p + 3)

    _block(sa_v, da_v, sb_v, db_v, ib, ia, NBLK)

    @pl.loop(0, BLK // RING - 1)
    def _grp_f(g):
        for q in range(RING):
            j = RING * g + q
            pltpu.make_async_copy(
                g_hbm.at[sb_v.at[pl.ds(j * CHUNK, CHUNK)]],
                rings[q], rsems[q]).wait()
            pltpu.sync_copy(
                rings[q], acc_sh.at[db_v.at[pl.ds(j * CHUNK, CHUNK)]],
                add=True)
            pltpu.async_copy(
                g_hbm.at[sb_v.at[pl.ds((j + RING) * CHUNK, CHUNK)]],
                rings[q], rsems[q])

    for q in range(RING):
        j = BLK - RING + q
        pltpu.make_async_copy(g_hbm.at[sb_v.at[pl.ds(j * CHUNK, CHUNK)]],
                              rings[q], rsems[q]).wait()
        pltpu.sync_copy(rings[q],
                        acc_sh.at[db_v.at[pl.ds(j * CHUNK, CHUNK)]], add=True)
    pltpu.make_async_copy(src_hbm.at[_iblk(0)], sa_v, ia).wait()
    pltpu.make_async_copy(dst_hbm.at[_iblk(0)], da_v, ia).wait()

    plsc.subcore_barrier()
    pltpu.sync_copy(acc_sh.at[pl.ds(s * RPT, RPT)],
                    out_hbm.at[c, pl.ds(s * RPT, RPT)])


@jax.jit
def _msg_call(g, srcR, dstR):
    mesh = plsc.VectorSubcoreMesh(core_axis_name="c", subcore_axis_name="s",
                                  num_cores=NC, num_subcores=NS)
    return pl.kernel(
        _msg_body,
        out_type=jax.ShapeDtypeStruct((NC, NPAD, D), jnp.float32),
        mesh=mesh,
        scratch_types=[
            pltpu.VMEM((IB,), jnp.int32),
            pltpu.VMEM((IB,), jnp.int32),
            pltpu.VMEM((IB,), jnp.int32),
            pltpu.VMEM((IB,), jnp.int32),
            pltpu.VMEM((CHUNK, D), jnp.float32),
            pltpu.VMEM((CHUNK, D), jnp.float32),
            pltpu.VMEM((CHUNK, D), jnp.float32),
            pltpu.VMEM((CHUNK, D), jnp.float32),
            pltpu.VMEM_SHARED((NPAD, D), jnp.float32),
            pltpu.SemaphoreType.DMA,
            pltpu.SemaphoreType.DMA,
            pltpu.SemaphoreType.DMA,
            pltpu.SemaphoreType.DMA,
            pltpu.SemaphoreType.DMA,
            pltpu.SemaphoreType.DMA,
        ],
    )(g, srcR, dstR)


def _tc_b_body(degp_ref, x_ref, w1_ref, g1_ref, norms_ref):
    deg = jnp.sum(degp_ref[...], axis=0)
    nrm = lax.rsqrt(jnp.where(deg > 0, deg, 1.0))
    norms_ref[...] = nrm
    ns = nrm[0][:, None]
    g1_ref[...] = jnp.dot(x_ref[...] * ns, w1_ref[...],
                          preferred_element_type=jnp.float32)


@jax.jit
def _tc_b_call(degp, x_pad, W1):
    return pl.pallas_call(
        _tc_b_body,
        grid=(GRID,),
        in_specs=[
            pl.BlockSpec((NW, 2, NB), lambda i: (0, 0, i)),
            pl.BlockSpec((NB, D), lambda i: (i, 0)),
            pl.BlockSpec((D, D), lambda i: (0, 0)),
        ],
        out_specs=[
            pl.BlockSpec((NB, D), lambda i: (i, 0)),
            pl.BlockSpec((2, NB), lambda i: (0, i)),
        ],
        out_shape=[
            jax.ShapeDtypeStruct((NPAD, D), jnp.float32),
            jax.ShapeDtypeStruct((2, NPAD), jnp.float32),
        ],
    )(degp, x_pad, W1)


def _tc_d_body(p_ref, norms_ref, b1_ref, w2_ref, g2_ref):
    i = pl.program_id(0)
    m = p_ref[0] + p_ref[1]
    nd = norms_ref[1][:, None]
    h = jnp.maximum(m * nd + b1_ref[...], 0.0)
    rows = i * NB + lax.broadcasted_iota(jnp.int32, (NB, 1), 0)
    h = jnp.where(rows < N, h, 0.0)
    ns = norms_ref[0][:, None]
    g2_ref[...] = jnp.dot(h * ns, w2_ref[...],
                          preferred_element_type=jnp.float32)


@jax.jit
def _tc_d_call(p, norms, b1r, W2):
    return pl.pallas_call(
        _tc_d_body,
        grid=(GRID,),
        in_specs=[
            pl.BlockSpec((NC, NB, D), lambda i: (0, i, 0)),
            pl.BlockSpec((2, NB), lambda i: (0, i)),
            pl.BlockSpec((1, D), lambda i: (0, 0)),
            pl.BlockSpec((D, D), lambda i: (0, 0)),
        ],
        out_specs=pl.BlockSpec((NB, D), lambda i: (i, 0)),
        out_shape=jax.ShapeDtypeStruct((NPAD, D), jnp.float32),
    )(p, norms, b1r, W2)


def _tc_f_body(p_ref, norms_ref, b2_ref, wg_ref, bg_ref, wo_ref, bo_ref,
               out_ref, m_s, s_s, r_v):
    i = pl.program_id(0)

    @pl.when(i == 0)
    def _init():
        m_s[0] = -jnp.inf
        s_s[0] = 0.0
        r_v[...] = jnp.zeros_like(r_v)

    m = p_ref[0] + p_ref[1]
    nd = norms_ref[1][:, None]
    h = jnp.maximum(m * nd + b2_ref[...], 0.0)
    rows = i * NB + lax.broadcasted_iota(jnp.int32, (NB, 1), 0)
    h = jnp.where(rows < N, h, 0.0)
    z = jnp.sum(h * wg_ref[...], axis=1, keepdims=True) + bg_ref[0, 0]
    z = jnp.where(rows < N, z, -jnp.inf)

    m_old = m_s[0]
    m_new = jnp.maximum(m_old, jnp.max(z))
    scale = jnp.exp(m_old - m_new)
    e = jnp.exp(z - m_new)
    s_s[0] = s_s[0] * scale + jnp.sum(e)
    r_v[...] = r_v[...] * scale + jnp.sum(e * h, axis=0, keepdims=True)
    m_s[0] = m_new

    @pl.when(i == pl.num_programs(0) - 1)
    def _fin():
        r = r_v[...] / s_s[0]
        val = jnp.sum(r * wo_ref[...]) + bo_ref[0, 0]
        out_ref[...] = jnp.full((8, 128), val, jnp.float32)


@jax.jit
def _tc_f_call(p, norms, b2r, wgr, bgr, wor, bor):
    return pl.pallas_call(
        _tc_f_body,
        grid=(GRID,),
        in_specs=[
            pl.BlockSpec((NC, NB, D), lambda i: (0, i, 0)),
            pl.BlockSpec((2, NB), lambda i: (0, i)),
            pl.BlockSpec((1, D), lambda i: (0, 0)),
            pl.BlockSpec((1, D), lambda i: (0, 0)),
            pl.BlockSpec((1, 1), lambda i: (0, 0)),
            pl.BlockSpec((1, D), lambda i: (0, 0)),
            pl.BlockSpec((1, 1), lambda i: (0, 0)),
        ],
        out_specs=pl.BlockSpec((8, 128), lambda i: (0, 0)),
        out_shape=jax.ShapeDtypeStruct((8, 128), jnp.float32),
        scratch_shapes=[
            pltpu.SMEM((1,), jnp.float32),
            pltpu.SMEM((1,), jnp.float32),
            pltpu.VMEM((1, D), jnp.float32),
        ],
    )(p, norms, b2r, wgr, bgr, wor, bor)


def kernel(x, edge_index, W1, b1, W2, b2, Wg, bg, Wo, bo):
    src = edge_index[0]
    dst = edge_index[1]
    pad = N + (jnp.arange(EPAD - E, dtype=jnp.int32) % (NPAD - N))
    src1 = jnp.concatenate([src, pad])
    dst1 = jnp.concatenate([dst, pad])
    dummy = jnp.full((NW, 1, IB), N, jnp.int32)
    srcR = jnp.concatenate([src1.reshape(NW, NBLK, IB), dummy],
                           axis=1).reshape(-1)
    dstR = jnp.concatenate([dst1.reshape(NW, NBLK, IB), dummy],
                           axis=1).reshape(-1)
    x_pad = jnp.zeros((NPAD, D), jnp.float32).at[:N].set(x)

    degp = _deg_call(src1, dst1)
    g1, norms = _tc_b_call(degp, x_pad, W1)
    p1 = _msg_call(g1, srcR, dstR)
    g2 = _tc_d_call(p1, norms, b1.reshape(1, D), W2)
    p2 = _msg_call(g2, srcR, dstR)
    buf = _tc_f_call(p2, norms, b2.reshape(1, D),
                     Wg[:, 0].reshape(1, D), bg.reshape(1, 1),
                     Wo[:, 0].reshape(1, D), bo.reshape(1, 1))
    return buf[:1, :1]
